# baseline (speedup 1.0000x reference)
import functools

import jax
import jax.numpy as jnp
from jax import lax
from jax.experimental import pallas as pl
from jax.experimental.pallas import tpu as pltpu
from jax.experimental.pallas import tpu_sc as plsc

N_NODES = 10000
N_PAD = 10240
N_EDGES = 160000
NC, NS = 2, 16
NW = NC * NS
EPW = N_EDGES // NW
CH = 128
NCHUNK = -(-EPW // CH)
EPWP = NCHUNK * CH
ZROWS = N_PAD // NS
TRASH = N_NODES + 8
EPS = 1e-5
BR = 1000
GRID = N_NODES // BR


def _sc_mesh():
    return plsc.VectorSubcoreMesh(
        core_axis_name="c", subcore_axis_name="s", num_cores=NC, num_subcores=NS
    )


@functools.partial(
    pl.kernel,
    out_type=jax.ShapeDtypeStruct((NC, N_PAD), jnp.float32),
    mesh=_sc_mesh(),
    scratch_types=[
        pltpu.VMEM((NCHUNK, CH), jnp.int32),
        pltpu.VMEM((CH,), jnp.float32),
        pltpu.VMEM_SHARED((N_PAD,), jnp.float32),
    ],
)
def _deg_kernel(dst_hbm, zeros_hbm, out_hbm, dstv, onesv, acc):
    c = lax.axis_index("c")
    s = lax.axis_index("s")
    wid = c * NS + s
    pltpu.sync_copy(zeros_hbm, acc.at[pl.ds(s * ZROWS, ZROWS)])
    pltpu.sync_copy(dst_hbm.at[wid], dstv)
    for i in range(CH // 16):
        onesv[pl.ds(i * 16, 16)] = jnp.ones((16,), jnp.float32)
    plsc.subcore_barrier()

    def body(j, carry):
        pltpu.sync_copy(onesv, acc.at[dstv.at[j]], add=True)
        return carry

    lax.fori_loop(0, NCHUNK, body, 0)
    plsc.subcore_barrier()
    pltpu.sync_copy(
        acc.at[pl.ds(s * ZROWS, ZROWS)], out_hbm.at[c, pl.ds(s * ZROWS, ZROWS)]
    )


def _make_scatter(d_feat):
    @functools.partial(
        pl.kernel,
        out_type=jax.ShapeDtypeStruct((NC, N_PAD, d_feat), jnp.float32),
        mesh=_sc_mesh(),
        scratch_types=[
            pltpu.VMEM((NCHUNK, CH), jnp.int32),
            pltpu.VMEM((NCHUNK, CH), jnp.int32),
            pltpu.VMEM((CH, d_feat), jnp.float32),
            pltpu.VMEM((CH, d_feat), jnp.float32),
            pltpu.VMEM_SHARED((N_PAD, d_feat), jnp.float32),
            pltpu.SemaphoreType.DMA,
            pltpu.SemaphoreType.DMA,
            pltpu.SemaphoreType.DMA,
            pltpu.SemaphoreType.DMA,
        ],
    )
    def _scat(y_hbm, src_hbm, dst_hbm, zeros_hbm, out_hbm, srcv, dstv, r0, r1,
              acc, g0, g1, s0, s1):
        rows = (r0, r1)
        gsem = (g0, g1)
        ssem = (s0, s1)
        c = lax.axis_index("c")
        s = lax.axis_index("s")
        wid = c * NS + s
        pltpu.sync_copy(zeros_hbm, acc.at[pl.ds(s * ZROWS, ZROWS)])
        pltpu.sync_copy(src_hbm.at[wid], srcv)
        pltpu.sync_copy(dst_hbm.at[wid], dstv)
        plsc.subcore_barrier()
        def wait_gather(j, b):
            pltpu.make_async_copy(y_hbm.at[srcv.at[j]], rows[b], gsem[b]).wait()

        def wait_scatter(j, b):
            pltpu.make_async_copy(rows[b], acc.at[dstv.at[j]], ssem[b]).wait()

        pltpu.async_copy(y_hbm.at[srcv.at[0]], rows[0], gsem[0])
        wait_gather(0, 0)
        pltpu.async_copy(rows[0], acc.at[dstv.at[0]], ssem[0], add=True)
        pltpu.async_copy(y_hbm.at[srcv.at[1]], rows[1], gsem[1])

        def step(j, b):
            wait_gather(j, b)
            pltpu.async_copy(rows[b], acc.at[dstv.at[j]], ssem[b], add=True)
            wait_scatter(j - 1, 1 - b)
            pltpu.async_copy(y_hbm.at[srcv.at[j + 1]], rows[1 - b], gsem[1 - b])

        def body(q, carry):
            step(2 * q + 1, 1)
            step(2 * q + 2, 0)
            return carry

        lax.fori_loop(0, (NCHUNK - 2) // 2, body, 0)
        j = NCHUNK - 1
        wait_gather(j, 1)
        pltpu.async_copy(rows[1], acc.at[dstv.at[j]], ssem[1], add=True)
        wait_scatter(j - 1, 0)
        wait_scatter(j, 1)
        plsc.subcore_barrier()
        pltpu.sync_copy(
            acc.at[pl.ds(s * ZROWS, ZROWS)], out_hbm.at[c, pl.ds(s * ZROWS, ZROWS)]
        )

    return _scat


_scatter128 = _make_scatter(128)


def _tc1_body(x_ref, w_ref, da_ref, db_ref, y_ref, dinv_ref):
    deg = da_ref[...] + db_ref[...] + 1.0
    dinv = 1.0 / jnp.sqrt(deg)
    y_ref[...] = (
        jnp.dot(x_ref[...], w_ref[...], preferred_element_type=jnp.float32) * dinv
    )
    dinv_ref[...] = dinv


def _tc_pre_body(acc_ref, y_ref, dinv_ref, b_ref, pre_ref, sum_ref, sq_ref):
    pre = (acc_ref[0] + acc_ref[1] + y_ref[...]) * dinv_ref[...] + b_ref[...]
    pre_ref[...] = pre
    ps = jnp.sum(pre, axis=0, keepdims=True)
    pq = jnp.sum(pre * pre, axis=0, keepdims=True)

    @pl.when(pl.program_id(0) == 0)
    def _():
        sum_ref[...] = ps
        sq_ref[...] = pq

    @pl.when(pl.program_id(0) > 0)
    def _():
        sum_ref[...] += ps
        sq_ref[...] += pq


def _bn_scale_shift(sum_ref, sq_ref, g_ref, bt_ref):
    mean = sum_ref[...] * (1.0 / N_NODES)
    var = sq_ref[...] * (1.0 / N_NODES) - mean * mean
    scale = g_ref[...] / jnp.sqrt(var + EPS)
    shift = bt_ref[...] - mean * scale
    return scale, shift


def _tc2c_body(sum_ref, sq_ref, g_ref, bt_ref, pre_ref, dinv_ref, w_ref, h_ref, y2_ref):
    scale, shift = _bn_scale_shift(sum_ref, sq_ref, g_ref, bt_ref)
    h = jnp.maximum(pre_ref[...] * scale + shift, 0.0)
    h_ref[...] = h
    y2 = jnp.dot(h, w_ref[...], preferred_element_type=jnp.float32) * dinv_ref[...]
    y2_ref[...] = jnp.concatenate([y2, jnp.zeros((BR, 64), jnp.float32)], axis=1)


def _tc4_body(sum_ref, sq_ref, g_ref, bt_ref, pre_ref, x_ref, h1_ref, wf_ref, bf_ref, out_ref):
    scale, shift = _bn_scale_shift(sum_ref, sq_ref, g_ref, bt_ref)
    h2 = jnp.maximum(pre_ref[...] * scale + shift, 0.0)[:, 0:64]
    o = (
        jnp.dot(x_ref[...], wf_ref[0:256, :], preferred_element_type=jnp.float32)
        + jnp.dot(h1_ref[...], wf_ref[256:384, :], preferred_element_type=jnp.float32)
        + jnp.dot(h2, wf_ref[384:448, :], preferred_element_type=jnp.float32)
        + bf_ref[...]
    )
    out_ref[...] = jnp.maximum(o, 0.0)


def _row_spec(d):
    return pl.BlockSpec((BR, d), lambda i: (i, 0))


def _full_spec(r, d):
    return pl.BlockSpec((r, d), lambda i: (0, 0))


def _tc1(x, w1, da, db):
    return pl.pallas_call(
        _tc1_body,
        grid=(GRID,),
        in_specs=[_row_spec(256), _full_spec(256, 128), _row_spec(1), _row_spec(1)],
        out_specs=[_row_spec(128), _row_spec(1)],
        out_shape=[
            jax.ShapeDtypeStruct((N_NODES, 128), jnp.float32),
            jax.ShapeDtypeStruct((N_NODES, 1), jnp.float32),
        ],
    )(x, w1, da, db)


def _tc_pre(acc, y, dinv, b, d):
    return pl.pallas_call(
        _tc_pre_body,
        grid=(GRID,),
        in_specs=[
            pl.BlockSpec((NC, BR, d), lambda i: (0, i, 0)),
            _row_spec(d),
            _row_spec(1),
            _full_spec(1, d),
        ],
        out_specs=[_row_spec(d), _full_spec(1, d), _full_spec(1, d)],
        out_shape=[
            jax.ShapeDtypeStruct((N_NODES, d), jnp.float32),
            jax.ShapeDtypeStruct((1, d), jnp.float32),
            jax.ShapeDtypeStruct((1, d), jnp.float32),
        ],
    )(acc, y, dinv, b)


def _tc2c(s1, q1, g, bt, pre, dinv, w2):
    return pl.pallas_call(
        _tc2c_body,
        grid=(GRID,),
        in_specs=[
            _full_spec(1, 128),
            _full_spec(1, 128),
            _full_spec(1, 128),
            _full_spec(1, 128),
            _row_spec(128),
            _row_spec(1),
            _full_spec(128, 64),
        ],
        out_specs=[_row_spec(128), _row_spec(128)],
        out_shape=[
            jax.ShapeDtypeStruct((N_NODES, 128), jnp.float32),
            jax.ShapeDtypeStruct((N_NODES, 128), jnp.float32),
        ],
    )(s1, q1, g, bt, pre, dinv, w2)


def _tc4(s2, q2, g, bt, pre2, x, h1, wf, bf):
    return pl.pallas_call(
        _tc4_body,
        grid=(GRID,),
        in_specs=[
            _full_spec(1, 128),
            _full_spec(1, 128),
            _full_spec(1, 128),
            _full_spec(1, 128),
            _row_spec(128),
            _row_spec(256),
            _row_spec(128),
            _full_spec(448, 256),
            _full_spec(1, 256),
        ],
        out_specs=_row_spec(256),
        out_shape=jax.ShapeDtypeStruct((N_NODES, 256), jnp.float32),
    )(s2, q2, g, bt, pre2, x, h1, wf, bf)


def kernel(node_features, edge_index, W1, b1, g1, bt1, W2, b2, g2, bt2, Wf, bf):
    x = node_features
    ei = edge_index.astype(jnp.int32)
    src3 = jnp.pad(ei[0].reshape(NW, EPW), ((0, 0), (0, EPWP - EPW))).reshape(
        NW, NCHUNK, CH
    )
    dst3 = jnp.pad(
        ei[1].reshape(NW, EPW), ((0, 0), (0, EPWP - EPW)), constant_values=TRASH
    ).reshape(NW, NCHUNK, CH)
    zd = jnp.zeros((ZROWS,), jnp.float32)
    z1 = jnp.zeros((ZROWS, 128), jnp.float32)

    degp = _deg_kernel(dst3, zd)
    da = degp[0, :N_NODES].reshape(N_NODES, 1)
    db = degp[1, :N_NODES].reshape(N_NODES, 1)

    y1, dinv = _tc1(x, W1, da, db)
    acc1 = _scatter128(y1, src3, dst3, z1)
    pre1, s1, q1 = _tc_pre(acc1, y1, dinv, b1.reshape(1, 128), 128)
    h1, y2 = _tc2c(s1, q1, g1.reshape(1, 128), bt1.reshape(1, 128), pre1, dinv, W2)
    acc2 = _scatter128(y2, src3, dst3, z1)
    pad64 = lambda v: jnp.pad(v, (0, 64)).reshape(1, 128)
    pre2, s2, q2 = _tc_pre(acc2, y2, dinv, pad64(b2), 128)
    return _tc4(
        s2, q2, pad64(g2), pad64(bt2), pre2, x, h1, Wf, bf.reshape(1, 256)
    )


# speedup vs baseline: 1.2283x; 1.2283x over previous
import functools

import jax
import jax.numpy as jnp
from jax import lax
from jax.experimental import pallas as pl
from jax.experimental.pallas import tpu as pltpu
from jax.experimental.pallas import tpu_sc as plsc

N_NODES = 10000
N_PAD = 10240
N_EDGES = 160000
NC, NS = 2, 16
NW = NC * NS
EPW = N_EDGES // NW
CH = 128
NCHUNK = -(-EPW // CH)
EPWP = NCHUNK * CH
ZROWS = N_PAD // NS
TRASH = N_NODES + 8
EPS = 1e-5
BR = 1000
GRID = N_NODES // BR


def _sc_mesh():
    return plsc.VectorSubcoreMesh(
        core_axis_name="c", subcore_axis_name="s", num_cores=NC, num_subcores=NS
    )


@functools.partial(
    pl.kernel,
    out_type=jax.ShapeDtypeStruct((NC, N_PAD), jnp.float32),
    mesh=_sc_mesh(),
    scratch_types=[
        pltpu.VMEM((NCHUNK, CH), jnp.int32),
        pltpu.VMEM((CH,), jnp.float32),
        pltpu.VMEM_SHARED((N_PAD,), jnp.float32),
    ],
)
def _deg_kernel(dst_hbm, zeros_hbm, out_hbm, dstv, onesv, acc):
    c = lax.axis_index("c")
    s = lax.axis_index("s")
    wid = c * NS + s
    pltpu.sync_copy(zeros_hbm, acc.at[pl.ds(s * ZROWS, ZROWS)])
    pltpu.sync_copy(dst_hbm.at[wid], dstv)
    for i in range(CH // 16):
        onesv[pl.ds(i * 16, 16)] = jnp.ones((16,), jnp.float32)
    plsc.subcore_barrier()

    def body(j, carry):
        pltpu.sync_copy(onesv, acc.at[dstv.at[j]], add=True)
        return carry

    lax.fori_loop(0, NCHUNK, body, 0)
    plsc.subcore_barrier()
    pltpu.sync_copy(
        acc.at[pl.ds(s * ZROWS, ZROWS)], out_hbm.at[c, pl.ds(s * ZROWS, ZROWS)]
    )


NPAIR = NCHUNK // 2


def _make_scatter(d_feat, tc_tiling=True):
    @functools.partial(
        pl.kernel,
        out_type=jax.ShapeDtypeStruct((NC, N_PAD, d_feat), jnp.float32),
        mesh=_sc_mesh(),
        compiler_params=pltpu.CompilerParams(use_tc_tiling_on_sc=tc_tiling),
        scratch_types=[
            pltpu.VMEM((NCHUNK, CH), jnp.int32),
            pltpu.VMEM((NCHUNK, CH), jnp.int32),
            pltpu.VMEM((CH, d_feat), jnp.float32),
            pltpu.VMEM((CH, d_feat), jnp.float32),
            pltpu.VMEM_SHARED((N_PAD, d_feat), jnp.float32),
            pltpu.SemaphoreType.DMA,
            pltpu.SemaphoreType.DMA,
        ],
    )
    def _scat(y_hbm, src_hbm, dst_hbm, zeros_hbm, out_hbm, srcv, dstv, rows0,
              rows1, acc, sem0, sem1):
        c = lax.axis_index("c")
        s = lax.axis_index("s")
        wid = c * NS + s
        pltpu.sync_copy(zeros_hbm, acc.at[pl.ds(s * ZROWS, ZROWS)])
        pltpu.sync_copy(src_hbm.at[wid], srcv)
        pltpu.sync_copy(dst_hbm.at[wid], dstv)
        plsc.subcore_barrier()
        pltpu.async_copy(y_hbm.at[srcv.at[0]], rows0, sem0)

        def body(p, carry):
            j0 = 2 * p
            pltpu.async_copy(y_hbm.at[srcv.at[j0 + 1]], rows1, sem1)
            pltpu.make_async_copy(y_hbm.at[srcv.at[j0]], rows0, sem0).wait()
            pltpu.sync_copy(rows0, acc.at[dstv.at[j0]], add=True)

            @pl.when(p < NPAIR - 1)
            def _():
                pltpu.async_copy(y_hbm.at[srcv.at[j0 + 2]], rows0, sem0)

            pltpu.make_async_copy(y_hbm.at[srcv.at[j0 + 1]], rows1, sem1).wait()
            pltpu.sync_copy(rows1, acc.at[dstv.at[j0 + 1]], add=True)
            return carry

        lax.fori_loop(0, NPAIR, body, 0)
        plsc.subcore_barrier()
        pltpu.sync_copy(
            acc.at[pl.ds(s * ZROWS, ZROWS)], out_hbm.at[c, pl.ds(s * ZROWS, ZROWS)]
        )

    return _scat


_scatter128 = _make_scatter(128)
_scatter64 = _make_scatter(64, tc_tiling=False)


def _tc1_body(x_ref, w_ref, da_ref, db_ref, y_ref, dinv_ref):
    deg = da_ref[...] + db_ref[...] + 1.0
    dinv = 1.0 / jnp.sqrt(deg)
    y_ref[...] = (
        jnp.dot(x_ref[...], w_ref[...], preferred_element_type=jnp.float32) * dinv
    )
    dinv_ref[...] = dinv


def _tc_pre_body(acc_ref, y_ref, dinv_ref, b_ref, pre_ref, sum_ref, sq_ref):
    pre = (acc_ref[0] + acc_ref[1] + y_ref[...]) * dinv_ref[...] + b_ref[...]
    pre_ref[...] = pre
    ps = jnp.sum(pre, axis=0, keepdims=True)
    pq = jnp.sum(pre * pre, axis=0, keepdims=True)

    @pl.when(pl.program_id(0) == 0)
    def _():
        sum_ref[...] = ps
        sq_ref[...] = pq

    @pl.when(pl.program_id(0) > 0)
    def _():
        sum_ref[...] += ps
        sq_ref[...] += pq


def _bn_scale_shift(sum_ref, sq_ref, g_ref, bt_ref):
    mean = sum_ref[...] * (1.0 / N_NODES)
    var = sq_ref[...] * (1.0 / N_NODES) - mean * mean
    scale = g_ref[...] / jnp.sqrt(var + EPS)
    shift = bt_ref[...] - mean * scale
    return scale, shift


def _tc2c_body(sum_ref, sq_ref, g_ref, bt_ref, pre_ref, dinv_ref, w_ref, h_ref, y2_ref):
    scale, shift = _bn_scale_shift(sum_ref, sq_ref, g_ref, bt_ref)
    h = jnp.maximum(pre_ref[...] * scale + shift, 0.0)
    h_ref[...] = h
    y2_ref[...] = (
        jnp.dot(h, w_ref[...], preferred_element_type=jnp.float32) * dinv_ref[...]
    )


def _tc4_body(sum_ref, sq_ref, g_ref, bt_ref, pre_ref, x_ref, h1_ref, wf_ref, bf_ref, out_ref):
    scale, shift = _bn_scale_shift(sum_ref, sq_ref, g_ref, bt_ref)
    h2 = jnp.maximum(pre_ref[...] * scale + shift, 0.0)
    o = (
        jnp.dot(x_ref[...], wf_ref[0:256, :], preferred_element_type=jnp.float32)
        + jnp.dot(h1_ref[...], wf_ref[256:384, :], preferred_element_type=jnp.float32)
        + jnp.dot(h2, wf_ref[384:448, :], preferred_element_type=jnp.float32)
        + bf_ref[...]
    )
    out_ref[...] = jnp.maximum(o, 0.0)


def _row_spec(d):
    return pl.BlockSpec((BR, d), lambda i: (i, 0))


def _full_spec(r, d):
    return pl.BlockSpec((r, d), lambda i: (0, 0))


def _tc1(x, w1, da, db):
    return pl.pallas_call(
        _tc1_body,
        grid=(GRID,),
        in_specs=[_row_spec(256), _full_spec(256, 128), _row_spec(1), _row_spec(1)],
        out_specs=[_row_spec(128), _row_spec(1)],
        out_shape=[
            jax.ShapeDtypeStruct((N_NODES, 128), jnp.float32),
            jax.ShapeDtypeStruct((N_NODES, 1), jnp.float32),
        ],
    )(x, w1, da, db)


def _tc_pre(acc, y, dinv, b, d):
    return pl.pallas_call(
        _tc_pre_body,
        grid=(GRID,),
        in_specs=[
            pl.BlockSpec((NC, BR, d), lambda i: (0, i, 0)),
            _row_spec(d),
            _row_spec(1),
            _full_spec(1, d),
        ],
        out_specs=[_row_spec(d), _full_spec(1, d), _full_spec(1, d)],
        out_shape=[
            jax.ShapeDtypeStruct((N_NODES, d), jnp.float32),
            jax.ShapeDtypeStruct((1, d), jnp.float32),
            jax.ShapeDtypeStruct((1, d), jnp.float32),
        ],
    )(acc, y, dinv, b)


def _tc2c(s1, q1, g, bt, pre, dinv, w2):
    return pl.pallas_call(
        _tc2c_body,
        grid=(GRID,),
        in_specs=[
            _full_spec(1, 128),
            _full_spec(1, 128),
            _full_spec(1, 128),
            _full_spec(1, 128),
            _row_spec(128),
            _row_spec(1),
            _full_spec(128, 64),
        ],
        out_specs=[_row_spec(128), _row_spec(64)],
        out_shape=[
            jax.ShapeDtypeStruct((N_NODES, 128), jnp.float32),
            jax.ShapeDtypeStruct((N_NODES, 64), jnp.float32),
        ],
    )(s1, q1, g, bt, pre, dinv, w2)


def _tc4(s2, q2, g, bt, pre2, x, h1, wf, bf):
    return pl.pallas_call(
        _tc4_body,
        grid=(GRID,),
        in_specs=[
            _full_spec(1, 64),
            _full_spec(1, 64),
            _full_spec(1, 64),
            _full_spec(1, 64),
            _row_spec(64),
            _row_spec(256),
            _row_spec(128),
            _full_spec(448, 256),
            _full_spec(1, 256),
        ],
        out_specs=_row_spec(256),
        out_shape=jax.ShapeDtypeStruct((N_NODES, 256), jnp.float32),
    )(s2, q2, g, bt, pre2, x, h1, wf, bf)


def kernel(node_features, edge_index, W1, b1, g1, bt1, W2, b2, g2, bt2, Wf, bf):
    x = node_features
    ei = edge_index.astype(jnp.int32)
    src3 = jnp.pad(ei[0].reshape(NW, EPW), ((0, 0), (0, EPWP - EPW))).reshape(
        NW, NCHUNK, CH
    )
    dst3 = jnp.pad(
        ei[1].reshape(NW, EPW), ((0, 0), (0, EPWP - EPW)), constant_values=TRASH
    ).reshape(NW, NCHUNK, CH)
    zd = jnp.zeros((ZROWS,), jnp.float32)
    z1 = jnp.zeros((ZROWS, 128), jnp.float32)
    z2 = jnp.zeros((ZROWS, 64), jnp.float32)

    degp = _deg_kernel(dst3, zd)
    da = degp[0, :N_NODES].reshape(N_NODES, 1)
    db = degp[1, :N_NODES].reshape(N_NODES, 1)

    y1, dinv = _tc1(x, W1, da, db)
    acc1 = _scatter128(y1, src3, dst3, z1)
    pre1, s1, q1 = _tc_pre(acc1, y1, dinv, b1.reshape(1, 128), 128)
    h1, y2 = _tc2c(s1, q1, g1.reshape(1, 128), bt1.reshape(1, 128), pre1, dinv, W2)
    acc2 = _scatter64(y2, src3, dst3, z2)
    pre2, s2, q2 = _tc_pre(acc2, y2, dinv, b2.reshape(1, 64), 64)
    return _tc4(
        s2, q2, g2.reshape(1, 64), bt2.reshape(1, 64), pre2, x, h1, Wf,
        bf.reshape(1, 256),
    )


# speedup vs baseline: 1.2284x; 1.0001x over previous
import functools

import jax
import jax.numpy as jnp
from jax import lax
from jax.experimental import pallas as pl
from jax.experimental.pallas import tpu as pltpu
from jax.experimental.pallas import tpu_sc as plsc

N_NODES = 10000
N_PAD = 10240
N_EDGES = 160000
NC, NS = 2, 16
NW = NC * NS
EPW = N_EDGES // NW
CH = 128
NCHUNK = -(-EPW // CH)
EPWP = NCHUNK * CH
ZROWS = N_PAD // NS
TRASH = N_NODES + 8
EPS = 1e-5
BR = 1000
GRID = N_NODES // BR


def _sc_mesh():
    return plsc.VectorSubcoreMesh(
        core_axis_name="c", subcore_axis_name="s", num_cores=NC, num_subcores=NS
    )


@functools.partial(
    pl.kernel,
    out_type=jax.ShapeDtypeStruct((NC, N_PAD), jnp.float32),
    mesh=_sc_mesh(),
    scratch_types=[
        pltpu.VMEM((NCHUNK, CH), jnp.int32),
        pltpu.VMEM((CH,), jnp.float32),
        pltpu.VMEM_SHARED((N_PAD,), jnp.float32),
    ],
)
def _deg_kernel(dst_hbm, zeros_hbm, out_hbm, dstv, onesv, acc):
    c = lax.axis_index("c")
    s = lax.axis_index("s")
    wid = c * NS + s
    pltpu.sync_copy(zeros_hbm, acc.at[pl.ds(s * ZROWS, ZROWS)])
    pltpu.sync_copy(dst_hbm.at[wid], dstv)
    for i in range(CH // 16):
        onesv[pl.ds(i * 16, 16)] = jnp.ones((16,), jnp.float32)
    plsc.subcore_barrier()

    def body(j, carry):
        pltpu.sync_copy(onesv, acc.at[dstv.at[j]], add=True)
        return carry

    lax.fori_loop(0, NCHUNK, body, 0)
    plsc.subcore_barrier()
    pltpu.sync_copy(
        acc.at[pl.ds(s * ZROWS, ZROWS)], out_hbm.at[c, pl.ds(s * ZROWS, ZROWS)]
    )


NPAIR = NCHUNK // 2


def _make_scatter(d_feat, tc_tiling=True):
    @functools.partial(
        pl.kernel,
        out_type=jax.ShapeDtypeStruct((NC, N_PAD, d_feat), jnp.float32),
        mesh=_sc_mesh(),
        compiler_params=pltpu.CompilerParams(use_tc_tiling_on_sc=tc_tiling),
        scratch_types=[
            pltpu.VMEM((NCHUNK, CH), jnp.int32),
            pltpu.VMEM((NCHUNK, CH), jnp.int32),
            pltpu.VMEM((CH, d_feat), jnp.float32),
            pltpu.VMEM((CH, d_feat), jnp.float32),
            pltpu.VMEM_SHARED((N_PAD, d_feat), jnp.float32),
            pltpu.SemaphoreType.DMA,
            pltpu.SemaphoreType.DMA,
        ],
    )
    def _scat(y_hbm, src_hbm, dst_hbm, zeros_hbm, out_hbm, srcv, dstv, rows0,
              rows1, acc, sem0, sem1):
        c = lax.axis_index("c")
        s = lax.axis_index("s")
        wid = c * NS + s
        pltpu.sync_copy(zeros_hbm, acc.at[pl.ds(s * ZROWS, ZROWS)])
        pltpu.sync_copy(src_hbm.at[wid], srcv)
        pltpu.sync_copy(dst_hbm.at[wid], dstv)
        plsc.subcore_barrier()
        pltpu.async_copy(y_hbm.at[srcv.at[0]], rows0, sem0)

        def body(p, carry):
            j0 = 2 * p
            pltpu.async_copy(y_hbm.at[srcv.at[j0 + 1]], rows1, sem1)
            pltpu.make_async_copy(y_hbm.at[srcv.at[j0]], rows0, sem0).wait()
            pltpu.sync_copy(rows0, acc.at[dstv.at[j0]], add=True)

            @pl.when(p < NPAIR - 1)
            def _():
                pltpu.async_copy(y_hbm.at[srcv.at[j0 + 2]], rows0, sem0)

            pltpu.make_async_copy(y_hbm.at[srcv.at[j0 + 1]], rows1, sem1).wait()
            pltpu.sync_copy(rows1, acc.at[dstv.at[j0 + 1]], add=True)
            return carry

        lax.fori_loop(0, NPAIR, body, 0)
        plsc.subcore_barrier()
        pltpu.sync_copy(
            acc.at[pl.ds(s * ZROWS, ZROWS)], out_hbm.at[c, pl.ds(s * ZROWS, ZROWS)]
        )

    return _scat


_scatter128 = _make_scatter(128, tc_tiling=False)
_scatter64 = _make_scatter(64, tc_tiling=False)


def _tc1_body(x_ref, w_ref, da_ref, db_ref, y_ref, dinv_ref):
    deg = da_ref[...] + db_ref[...] + 1.0
    dinv = 1.0 / jnp.sqrt(deg)
    y_ref[...] = (
        jnp.dot(x_ref[...], w_ref[...], preferred_element_type=jnp.float32) * dinv
    )
    dinv_ref[...] = dinv


def _tc_pre_body(acc_ref, y_ref, dinv_ref, b_ref, pre_ref, sum_ref, sq_ref):
    pre = (acc_ref[0] + acc_ref[1] + y_ref[...]) * dinv_ref[...] + b_ref[...]
    pre_ref[...] = pre
    ps = jnp.sum(pre, axis=0, keepdims=True)
    pq = jnp.sum(pre * pre, axis=0, keepdims=True)

    @pl.when(pl.program_id(0) == 0)
    def _():
        sum_ref[...] = ps
        sq_ref[...] = pq

    @pl.when(pl.program_id(0) > 0)
    def _():
        sum_ref[...] += ps
        sq_ref[...] += pq


def _bn_scale_shift(sum_ref, sq_ref, g_ref, bt_ref):
    mean = sum_ref[...] * (1.0 / N_NODES)
    var = sq_ref[...] * (1.0 / N_NODES) - mean * mean
    scale = g_ref[...] / jnp.sqrt(var + EPS)
    shift = bt_ref[...] - mean * scale
    return scale, shift


def _tc2c_body(sum_ref, sq_ref, g_ref, bt_ref, pre_ref, dinv_ref, w_ref, h_ref, y2_ref):
    scale, shift = _bn_scale_shift(sum_ref, sq_ref, g_ref, bt_ref)
    h = jnp.maximum(pre_ref[...] * scale + shift, 0.0)
    h_ref[...] = h
    y2_ref[...] = (
        jnp.dot(h, w_ref[...], preferred_element_type=jnp.float32) * dinv_ref[...]
    )


def _tc4_body(sum_ref, sq_ref, g_ref, bt_ref, pre_ref, x_ref, h1_ref, wf_ref, bf_ref, out_ref):
    scale, shift = _bn_scale_shift(sum_ref, sq_ref, g_ref, bt_ref)
    h2 = jnp.maximum(pre_ref[...] * scale + shift, 0.0)
    o = (
        jnp.dot(x_ref[...], wf_ref[0:256, :], preferred_element_type=jnp.float32)
        + jnp.dot(h1_ref[...], wf_ref[256:384, :], preferred_element_type=jnp.float32)
        + jnp.dot(h2, wf_ref[384:448, :], preferred_element_type=jnp.float32)
        + bf_ref[...]
    )
    out_ref[...] = jnp.maximum(o, 0.0)


def _row_spec(d):
    return pl.BlockSpec((BR, d), lambda i: (i, 0))


def _full_spec(r, d):
    return pl.BlockSpec((r, d), lambda i: (0, 0))


def _tc1(x, w1, da, db):
    return pl.pallas_call(
        _tc1_body,
        grid=(GRID,),
        in_specs=[_row_spec(256), _full_spec(256, 128), _row_spec(1), _row_spec(1)],
        out_specs=[_row_spec(128), _row_spec(1)],
        out_shape=[
            jax.ShapeDtypeStruct((N_NODES, 128), jnp.float32),
            jax.ShapeDtypeStruct((N_NODES, 1), jnp.float32),
        ],
    )(x, w1, da, db)


def _tc_pre(acc, y, dinv, b, d):
    return pl.pallas_call(
        _tc_pre_body,
        grid=(GRID,),
        in_specs=[
            pl.BlockSpec((NC, BR, d), lambda i: (0, i, 0)),
            _row_spec(d),
            _row_spec(1),
            _full_spec(1, d),
        ],
        out_specs=[_row_spec(d), _full_spec(1, d), _full_spec(1, d)],
        out_shape=[
            jax.ShapeDtypeStruct((N_NODES, d), jnp.float32),
            jax.ShapeDtypeStruct((1, d), jnp.float32),
            jax.ShapeDtypeStruct((1, d), jnp.float32),
        ],
    )(acc, y, dinv, b)


def _tc2c(s1, q1, g, bt, pre, dinv, w2):
    return pl.pallas_call(
        _tc2c_body,
        grid=(GRID,),
        in_specs=[
            _full_spec(1, 128),
            _full_spec(1, 128),
            _full_spec(1, 128),
            _full_spec(1, 128),
            _row_spec(128),
            _row_spec(1),
            _full_spec(128, 64),
        ],
        out_specs=[_row_spec(128), _row_spec(64)],
        out_shape=[
            jax.ShapeDtypeStruct((N_NODES, 128), jnp.float32),
            jax.ShapeDtypeStruct((N_NODES, 64), jnp.float32),
        ],
    )(s1, q1, g, bt, pre, dinv, w2)


def _tc4(s2, q2, g, bt, pre2, x, h1, wf, bf):
    return pl.pallas_call(
        _tc4_body,
        grid=(GRID,),
        in_specs=[
            _full_spec(1, 64),
            _full_spec(1, 64),
            _full_spec(1, 64),
            _full_spec(1, 64),
            _row_spec(64),
            _row_spec(256),
            _row_spec(128),
            _full_spec(448, 256),
            _full_spec(1, 256),
        ],
        out_specs=_row_spec(256),
        out_shape=jax.ShapeDtypeStruct((N_NODES, 256), jnp.float32),
    )(s2, q2, g, bt, pre2, x, h1, wf, bf)


def kernel(node_features, edge_index, W1, b1, g1, bt1, W2, b2, g2, bt2, Wf, bf):
    x = node_features
    ei = edge_index.astype(jnp.int32)
    src3 = jnp.pad(ei[0].reshape(NW, EPW), ((0, 0), (0, EPWP - EPW))).reshape(
        NW, NCHUNK, CH
    )
    dst3 = jnp.pad(
        ei[1].reshape(NW, EPW), ((0, 0), (0, EPWP - EPW)), constant_values=TRASH
    ).reshape(NW, NCHUNK, CH)
    zd = jnp.zeros((ZROWS,), jnp.float32)
    z1 = jnp.zeros((ZROWS, 128), jnp.float32)
    z2 = jnp.zeros((ZROWS, 64), jnp.float32)

    degp = _deg_kernel(dst3, zd)
    da = degp[0, :N_NODES].reshape(N_NODES, 1)
    db = degp[1, :N_NODES].reshape(N_NODES, 1)

    y1, dinv = _tc1(x, W1, da, db)
    acc1 = _scatter128(y1, src3, dst3, z1)
    pre1, s1, q1 = _tc_pre(acc1, y1, dinv, b1.reshape(1, 128), 128)
    h1, y2 = _tc2c(s1, q1, g1.reshape(1, 128), bt1.reshape(1, 128), pre1, dinv, W2)
    acc2 = _scatter64(y2, src3, dst3, z2)
    pre2, s2, q2 = _tc_pre(acc2, y2, dinv, b2.reshape(1, 64), 64)
    return _tc4(
        s2, q2, g2.reshape(1, 64), bt2.reshape(1, 64), pre2, x, h1, Wf,
        bf.reshape(1, 256),
    )


# speedup vs baseline: 1.6179x; 1.3171x over previous
import functools

import jax
import jax.numpy as jnp
from jax import lax
from jax.experimental import pallas as pl
from jax.experimental.pallas import tpu as pltpu
from jax.experimental.pallas import tpu_sc as plsc

N_NODES = 10000
N_PAD = 10240
N_EDGES = 160000
NC, NS = 2, 16
NW = NC * NS
EPW = N_EDGES // NW
CH = 128
NCHUNK = -(-EPW // CH)
EPWP = NCHUNK * CH
ZROWS = N_PAD // NS
TRASH = N_NODES + 8
EPS = 1e-5
BR = 1000
GRID = N_NODES // BR


def _sc_mesh():
    return plsc.VectorSubcoreMesh(
        core_axis_name="c", subcore_axis_name="s", num_cores=NC, num_subcores=NS
    )


@functools.partial(
    pl.kernel,
    out_type=jax.ShapeDtypeStruct((NC, N_PAD), jnp.float32),
    mesh=_sc_mesh(),
    scratch_types=[
        pltpu.VMEM((NCHUNK, CH), jnp.int32),
        pltpu.VMEM((CH,), jnp.float32),
        pltpu.VMEM_SHARED((N_PAD,), jnp.float32),
    ],
)
def _deg_kernel(dst_hbm, zeros_hbm, out_hbm, dstv, onesv, acc):
    c = lax.axis_index("c")
    s = lax.axis_index("s")
    wid = c * NS + s
    pltpu.sync_copy(zeros_hbm, acc.at[pl.ds(s * ZROWS, ZROWS)])
    pltpu.sync_copy(dst_hbm.at[wid], dstv)
    for i in range(CH // 16):
        onesv[pl.ds(i * 16, 16)] = jnp.ones((16,), jnp.float32)
    plsc.subcore_barrier()

    def body(j, carry):
        pltpu.sync_copy(onesv, acc.at[dstv.at[j]], add=True)
        return carry

    lax.fori_loop(0, NCHUNK, body, 0)
    plsc.subcore_barrier()
    pltpu.sync_copy(
        acc.at[pl.ds(s * ZROWS, ZROWS)], out_hbm.at[c, pl.ds(s * ZROWS, ZROWS)]
    )


NPAIR = NCHUNK // 2


def _make_scatter(d_feat, dtype=jnp.float32):
    @functools.partial(
        pl.kernel,
        out_type=jax.ShapeDtypeStruct((NC, N_PAD, d_feat), dtype),
        mesh=_sc_mesh(),
        compiler_params=pltpu.CompilerParams(use_tc_tiling_on_sc=False),
        scratch_types=[
            pltpu.VMEM((NCHUNK, CH), jnp.int32),
            pltpu.VMEM((NCHUNK, CH), jnp.int32),
            pltpu.VMEM((CH, d_feat), dtype),
            pltpu.VMEM((CH, d_feat), dtype),
            pltpu.VMEM_SHARED((N_PAD, d_feat), dtype),
            pltpu.SemaphoreType.DMA,
            pltpu.SemaphoreType.DMA,
        ],
    )
    def _scat(y_hbm, src_hbm, dst_hbm, zeros_hbm, out_hbm, srcv, dstv, rows0,
              rows1, acc, sem0, sem1):
        c = lax.axis_index("c")
        s = lax.axis_index("s")
        wid = c * NS + s
        pltpu.sync_copy(zeros_hbm, acc.at[pl.ds(s * ZROWS, ZROWS)])
        pltpu.sync_copy(src_hbm.at[wid], srcv)
        pltpu.sync_copy(dst_hbm.at[wid], dstv)
        plsc.subcore_barrier()
        pltpu.async_copy(y_hbm.at[srcv.at[0]], rows0, sem0)

        def body(p, carry):
            j0 = 2 * p
            pltpu.async_copy(y_hbm.at[srcv.at[j0 + 1]], rows1, sem1)
            pltpu.make_async_copy(y_hbm.at[srcv.at[j0]], rows0, sem0).wait()
            pltpu.sync_copy(rows0, acc.at[dstv.at[j0]], add=True)

            @pl.when(p < NPAIR - 1)
            def _():
                pltpu.async_copy(y_hbm.at[srcv.at[j0 + 2]], rows0, sem0)

            pltpu.make_async_copy(y_hbm.at[srcv.at[j0 + 1]], rows1, sem1).wait()
            pltpu.sync_copy(rows1, acc.at[dstv.at[j0 + 1]], add=True)
            return carry

        lax.fori_loop(0, NPAIR, body, 0)
        plsc.subcore_barrier()
        pltpu.sync_copy(
            acc.at[pl.ds(s * ZROWS, ZROWS)], out_hbm.at[c, pl.ds(s * ZROWS, ZROWS)]
        )

    return _scat


_scatter128 = _make_scatter(128, jnp.bfloat16)
_scatter64 = _make_scatter(64, jnp.bfloat16)


def _tc1_body(x_ref, w_ref, da_ref, db_ref, y_ref, y16_ref, dinv_ref):
    deg = da_ref[...] + db_ref[...] + 1.0
    dinv = 1.0 / jnp.sqrt(deg)
    y = jnp.dot(x_ref[...], w_ref[...], preferred_element_type=jnp.float32) * dinv
    y_ref[...] = y
    y16_ref[...] = y.astype(jnp.bfloat16)
    dinv_ref[...] = dinv


def _tc_pre_body(acc_ref, y_ref, dinv_ref, b_ref, pre_ref, sum_ref, sq_ref):
    a = acc_ref[0].astype(jnp.float32) + acc_ref[1].astype(jnp.float32)
    pre = (a + y_ref[...]) * dinv_ref[...] + b_ref[...]
    pre_ref[...] = pre
    ps = jnp.sum(pre, axis=0, keepdims=True)
    pq = jnp.sum(pre * pre, axis=0, keepdims=True)

    @pl.when(pl.program_id(0) == 0)
    def _():
        sum_ref[...] = ps
        sq_ref[...] = pq

    @pl.when(pl.program_id(0) > 0)
    def _():
        sum_ref[...] += ps
        sq_ref[...] += pq


def _bn_scale_shift(sum_ref, sq_ref, g_ref, bt_ref):
    mean = sum_ref[...] * (1.0 / N_NODES)
    var = sq_ref[...] * (1.0 / N_NODES) - mean * mean
    scale = g_ref[...] / jnp.sqrt(var + EPS)
    shift = bt_ref[...] - mean * scale
    return scale, shift


def _tc2c_body(sum_ref, sq_ref, g_ref, bt_ref, pre_ref, dinv_ref, w_ref, h_ref,
               y2_ref, y216_ref):
    scale, shift = _bn_scale_shift(sum_ref, sq_ref, g_ref, bt_ref)
    h = jnp.maximum(pre_ref[...] * scale + shift, 0.0)
    h_ref[...] = h
    y2 = jnp.dot(h, w_ref[...], preferred_element_type=jnp.float32) * dinv_ref[...]
    y2_ref[...] = y2
    y216_ref[...] = y2.astype(jnp.bfloat16)


def _tc4_body(sum_ref, sq_ref, g_ref, bt_ref, pre_ref, x_ref, h1_ref, wf_ref, bf_ref, out_ref):
    scale, shift = _bn_scale_shift(sum_ref, sq_ref, g_ref, bt_ref)
    h2 = jnp.maximum(pre_ref[...] * scale + shift, 0.0)
    o = (
        jnp.dot(x_ref[...], wf_ref[0:256, :], preferred_element_type=jnp.float32)
        + jnp.dot(h1_ref[...], wf_ref[256:384, :], preferred_element_type=jnp.float32)
        + jnp.dot(h2, wf_ref[384:448, :], preferred_element_type=jnp.float32)
        + bf_ref[...]
    )
    out_ref[...] = jnp.maximum(o, 0.0)


def _row_spec(d):
    return pl.BlockSpec((BR, d), lambda i: (i, 0))


def _full_spec(r, d):
    return pl.BlockSpec((r, d), lambda i: (0, 0))


def _tc1(x, w1, da, db):
    return pl.pallas_call(
        _tc1_body,
        grid=(GRID,),
        in_specs=[_row_spec(256), _full_spec(256, 128), _row_spec(1), _row_spec(1)],
        out_specs=[_row_spec(128), _row_spec(128), _row_spec(1)],
        out_shape=[
            jax.ShapeDtypeStruct((N_NODES, 128), jnp.float32),
            jax.ShapeDtypeStruct((N_NODES, 128), jnp.bfloat16),
            jax.ShapeDtypeStruct((N_NODES, 1), jnp.float32),
        ],
    )(x, w1, da, db)


def _tc_pre(acc, y, dinv, b, d):
    return pl.pallas_call(
        _tc_pre_body,
        grid=(GRID,),
        in_specs=[
            pl.BlockSpec((NC, BR, d), lambda i: (0, i, 0)),
            _row_spec(d),
            _row_spec(1),
            _full_spec(1, d),
        ],
        out_specs=[_row_spec(d), _full_spec(1, d), _full_spec(1, d)],
        out_shape=[
            jax.ShapeDtypeStruct((N_NODES, d), jnp.float32),
            jax.ShapeDtypeStruct((1, d), jnp.float32),
            jax.ShapeDtypeStruct((1, d), jnp.float32),
        ],
    )(acc, y, dinv, b)


def _tc2c(s1, q1, g, bt, pre, dinv, w2):
    return pl.pallas_call(
        _tc2c_body,
        grid=(GRID,),
        in_specs=[
            _full_spec(1, 128),
            _full_spec(1, 128),
            _full_spec(1, 128),
            _full_spec(1, 128),
            _row_spec(128),
            _row_spec(1),
            _full_spec(128, 64),
        ],
        out_specs=[_row_spec(128), _row_spec(64), _row_spec(64)],
        out_shape=[
            jax.ShapeDtypeStruct((N_NODES, 128), jnp.float32),
            jax.ShapeDtypeStruct((N_NODES, 64), jnp.float32),
            jax.ShapeDtypeStruct((N_NODES, 64), jnp.bfloat16),
        ],
    )(s1, q1, g, bt, pre, dinv, w2)


def _tc4(s2, q2, g, bt, pre2, x, h1, wf, bf):
    return pl.pallas_call(
        _tc4_body,
        grid=(GRID,),
        in_specs=[
            _full_spec(1, 64),
            _full_spec(1, 64),
            _full_spec(1, 64),
            _full_spec(1, 64),
            _row_spec(64),
            _row_spec(256),
            _row_spec(128),
            _full_spec(448, 256),
            _full_spec(1, 256),
        ],
        out_specs=_row_spec(256),
        out_shape=jax.ShapeDtypeStruct((N_NODES, 256), jnp.float32),
    )(s2, q2, g, bt, pre2, x, h1, wf, bf)


def kernel(node_features, edge_index, W1, b1, g1, bt1, W2, b2, g2, bt2, Wf, bf):
    x = node_features
    ei = edge_index.astype(jnp.int32)
    src3 = jnp.pad(ei[0].reshape(NW, EPW), ((0, 0), (0, EPWP - EPW))).reshape(
        NW, NCHUNK, CH
    )
    dst3 = jnp.pad(
        ei[1].reshape(NW, EPW), ((0, 0), (0, EPWP - EPW)), constant_values=TRASH
    ).reshape(NW, NCHUNK, CH)
    zd = jnp.zeros((ZROWS,), jnp.float32)
    z1 = jnp.zeros((ZROWS, 128), jnp.bfloat16)
    z2 = jnp.zeros((ZROWS, 64), jnp.bfloat16)

    degp = _deg_kernel(dst3, zd)
    da = degp[0, :N_NODES].reshape(N_NODES, 1)
    db = degp[1, :N_NODES].reshape(N_NODES, 1)

    y1, y1b, dinv = _tc1(x, W1, da, db)
    acc1 = _scatter128(y1b, src3, dst3, z1)
    pre1, s1, q1 = _tc_pre(acc1, y1, dinv, b1.reshape(1, 128), 128)
    h1, y2, y2b = _tc2c(s1, q1, g1.reshape(1, 128), bt1.reshape(1, 128), pre1, dinv, W2)
    acc2 = _scatter64(y2b, src3, dst3, z2)
    pre2, s2, q2 = _tc_pre(acc2, y2, dinv, b2.reshape(1, 64), 64)
    return _tc4(
        s2, q2, g2.reshape(1, 64), bt2.reshape(1, 64), pre2, x, h1, Wf,
        bf.reshape(1, 256),
    )
